# baseline (speedup 1.0000x reference)
import functools

import jax
import jax.numpy as jnp
from jax import lax
from jax.experimental import pallas as pl
from jax.experimental.pallas import tpu as pltpu
from jax.experimental.pallas import tpu_sc as plsc

N = 10000
D = 128
NPAD = 10240
E = 320000
NW = 32
CHUNK = 128
CPW = 80
EPW = CPW * CHUNK
EPAD = NW * EPW
ROWS_PT = NPAD // 16
BLK = 512
GRID = NPAD // BLK

_mesh = functools.partial(
    plsc.VectorSubcoreMesh, core_axis_name="c", subcore_axis_name="s")


def _deg_body(dst_hbm, ones_hbm, z128_hbm, out_hbm, dstv, onev, acc):
    c = lax.axis_index("c")
    s = lax.axis_index("s")
    wid = c * 16 + s
    pltpu.sync_copy(z128_hbm, acc.at[pl.ds(s * ROWS_PT, ROWS_PT)])
    pltpu.sync_copy(dst_hbm.at[pl.ds(wid * CPW, CPW)], dstv)
    pltpu.sync_copy(ones_hbm, onev)
    plsc.subcore_barrier()

    def body(j, carry):
        pltpu.sync_copy(onev, acc.at[dstv.at[j]], add=True)
        return carry

    lax.fori_loop(0, CPW, body, 0)
    plsc.subcore_barrier()
    pltpu.sync_copy(acc.at[pl.ds(s * ROWS_PT, ROWS_PT)],
                    out_hbm.at[c, pl.ds(s * ROWS_PT, ROWS_PT)])


def _deg_sc(dst2d, ones128, z128):
    return pl.kernel(
        _deg_body,
        out_type=jax.ShapeDtypeStruct((2, NPAD, D), jnp.float32),
        mesh=_mesh(),
        scratch_types=[
            pltpu.VMEM((CPW, CHUNK), jnp.int32),
            pltpu.VMEM((CHUNK, D), jnp.float32),
            pltpu.VMEM_SHARED((NPAD, D), jnp.float32),
        ],
    )(dst2d, ones128, z128)


def _edge_body(hs_hbm, src_hbm, dst_hbm, z128_hbm, out_hbm,
               srcv, dstv, gbuf, acc):
    c = lax.axis_index("c")
    s = lax.axis_index("s")
    wid = c * 16 + s
    pltpu.sync_copy(z128_hbm, acc.at[pl.ds(s * ROWS_PT, ROWS_PT)])
    pltpu.sync_copy(src_hbm.at[pl.ds(wid * CPW, CPW)], srcv)
    pltpu.sync_copy(dst_hbm.at[pl.ds(wid * CPW, CPW)], dstv)
    plsc.subcore_barrier()

    def body(j, carry):
        pltpu.sync_copy(hs_hbm.at[srcv.at[j]], gbuf)
        pltpu.sync_copy(gbuf, acc.at[dstv.at[j]], add=True)
        return carry

    lax.fori_loop(0, CPW, body, 0)
    plsc.subcore_barrier()
    pltpu.sync_copy(acc.at[pl.ds(s * ROWS_PT, ROWS_PT)],
                    out_hbm.at[c, pl.ds(s * ROWS_PT, ROWS_PT)])


def _edge_sc(hs, src2d, dst2d, z128):
    return pl.kernel(
        _edge_body,
        out_type=jax.ShapeDtypeStruct((2, NPAD, D), jnp.float32),
        mesh=_mesh(),
        scratch_types=[
            pltpu.VMEM((CPW, CHUNK), jnp.int32),
            pltpu.VMEM((CPW, CHUNK), jnp.int32),
            pltpu.VMEM((CHUNK, D), jnp.float32),
            pltpu.VMEM_SHARED((NPAD, D), jnp.float32),
        ],
    )(hs, src2d, dst2d, z128)


def _prep_body(x_ref, w_ref, d0_ref, d1_ref, hs_ref, di_ref):
    deg = d0_ref[...] + d1_ref[...] + 1.0
    di = lax.rsqrt(deg)
    hs_ref[...] = jnp.dot(x_ref[...], w_ref[...],
                          preferred_element_type=jnp.float32) * di
    di_ref[...] = di


def _prep_tc(x, w1, d0, d1):
    return pl.pallas_call(
        _prep_body,
        grid=(GRID,),
        in_specs=[
            pl.BlockSpec((BLK, D), lambda i: (i, 0)),
            pl.BlockSpec((D, D), lambda i: (0, 0)),
            pl.BlockSpec((BLK, D), lambda i: (i, 0)),
            pl.BlockSpec((BLK, D), lambda i: (i, 0)),
        ],
        out_specs=[
            pl.BlockSpec((BLK, D), lambda i: (i, 0)),
            pl.BlockSpec((BLK, D), lambda i: (i, 0)),
        ],
        out_shape=[
            jax.ShapeDtypeStruct((NPAD, D), jnp.float32),
            jax.ShapeDtypeStruct((NPAD, D), jnp.float32),
        ],
    )(x, w1, d0, d1)


def _mid_body(a0_ref, a1_ref, hs_ref, di_ref, b_ref, w_ref, out_ref):
    di = di_ref[...]
    h = di * (a0_ref[...] + a1_ref[...] + hs_ref[...]) + b_ref[...]
    h = jnp.maximum(h, 0.0)
    out_ref[...] = jnp.dot(h, w_ref[...],
                           preferred_element_type=jnp.float32) * di


def _mid_tc(a0, a1, hs, di, b1, w2):
    return pl.pallas_call(
        _mid_body,
        grid=(GRID,),
        in_specs=[
            pl.BlockSpec((BLK, D), lambda i: (i, 0)),
            pl.BlockSpec((BLK, D), lambda i: (i, 0)),
            pl.BlockSpec((BLK, D), lambda i: (i, 0)),
            pl.BlockSpec((BLK, D), lambda i: (i, 0)),
            pl.BlockSpec((1, D), lambda i: (0, 0)),
            pl.BlockSpec((D, D), lambda i: (0, 0)),
        ],
        out_specs=pl.BlockSpec((BLK, D), lambda i: (i, 0)),
        out_shape=jax.ShapeDtypeStruct((NPAD, D), jnp.float32),
    )(a0, a1, hs, di, b1, w2)


def _final_body(a0_ref, a1_ref, hs_ref, di_ref, b_ref, h_ref, p_ref):
    i = pl.program_id(0)
    h = di_ref[...] * (a0_ref[...] + a1_ref[...] + hs_ref[...]) + b_ref[...]
    h = jnp.maximum(h, 0.0)
    h_ref[...] = h
    row = lax.broadcasted_iota(jnp.int32, (BLK, D), 0) + i * BLK
    hm = jnp.where(row < N, h, -jnp.inf)
    bm = jnp.max(hm, axis=0, keepdims=True)

    @pl.when(i == 0)
    def _():
        p_ref[...] = bm

    @pl.when(i > 0)
    def _():
        p_ref[...] = jnp.maximum(p_ref[...], bm)


def _final_tc(a0, a1, hs, di, b2):
    return pl.pallas_call(
        _final_body,
        grid=(GRID,),
        in_specs=[
            pl.BlockSpec((BLK, D), lambda i: (i, 0)),
            pl.BlockSpec((BLK, D), lambda i: (i, 0)),
            pl.BlockSpec((BLK, D), lambda i: (i, 0)),
            pl.BlockSpec((BLK, D), lambda i: (i, 0)),
            pl.BlockSpec((1, D), lambda i: (0, 0)),
        ],
        out_specs=[
            pl.BlockSpec((BLK, D), lambda i: (i, 0)),
            pl.BlockSpec((1, D), lambda i: (0, 0)),
        ],
        out_shape=[
            jax.ShapeDtypeStruct((NPAD, D), jnp.float32),
            jax.ShapeDtypeStruct((1, D), jnp.float32),
        ],
    )(a0, a1, hs, di, b2)


def kernel(x, edge_index, W1, b1, W2, b2):
    src = edge_index[0].astype(jnp.int32)
    dst = edge_index[1].astype(jnp.int32)
    pad = (N + (jnp.arange(EPAD - E, dtype=jnp.int32) % (NPAD - N)))
    src2d = jnp.concatenate([src, pad]).reshape(NW * CPW, CHUNK)
    dst2d = jnp.concatenate([dst, pad]).reshape(NW * CPW, CHUNK)
    xp = jnp.zeros((NPAD, D), jnp.float32).at[:N].set(x)

    ones128 = jnp.ones((CHUNK, D), jnp.float32)
    z128 = jnp.zeros((ROWS_PT, D), jnp.float32)
    b1r = b1.reshape(1, D)
    b2r = b2.reshape(1, D)

    degs = _deg_sc(dst2d, ones128, z128)
    hs1, dinv = _prep_tc(xp, W1, degs[0], degs[1])
    acc1 = _edge_sc(hs1, src2d, dst2d, z128)
    hs2 = _mid_tc(acc1[0], acc1[1], hs1, dinv, b1r, W2)
    acc2 = _edge_sc(hs2, src2d, dst2d, z128)
    h2, p = _final_tc(acc2[0], acc2[1], hs2, dinv, b2r)
    return (h2[:N], p)


# speedup vs baseline: 1.1713x; 1.1713x over previous
import functools

import jax
import jax.numpy as jnp
from jax import lax
from jax.experimental import pallas as pl
from jax.experimental.pallas import tpu as pltpu
from jax.experimental.pallas import tpu_sc as plsc

N = 10000
D = 128
NPAD = 10240
E = 320000
NW = 32
CHUNK = 128
CPW = 80
EPW = CPW * CHUNK
EPAD = NW * EPW
ROWS_PT = NPAD // 16
BLK = 512
GRID = NPAD // BLK

_mesh = functools.partial(
    plsc.VectorSubcoreMesh, core_axis_name="c", subcore_axis_name="s")


def _deg_body(dst_hbm, ones_hbm, z128_hbm, out_hbm, dstv, onev, acc):
    c = lax.axis_index("c")
    s = lax.axis_index("s")
    wid = c * 16 + s
    pltpu.sync_copy(z128_hbm, acc.at[pl.ds(s * ROWS_PT, ROWS_PT)])
    pltpu.sync_copy(dst_hbm.at[pl.ds(wid * CPW, CPW)], dstv)
    pltpu.sync_copy(ones_hbm, onev)
    plsc.subcore_barrier()

    def body(j, carry):
        pltpu.sync_copy(onev, acc.at[dstv.at[j]], add=True)
        return carry

    lax.fori_loop(0, CPW, body, 0)
    plsc.subcore_barrier()
    pltpu.sync_copy(acc.at[pl.ds(s * ROWS_PT, ROWS_PT)],
                    out_hbm.at[c, pl.ds(s * ROWS_PT, ROWS_PT)])


def _deg_sc(dst2d, ones128, z128):
    return pl.kernel(
        _deg_body,
        out_type=jax.ShapeDtypeStruct((2, NPAD, D), jnp.float32),
        mesh=_mesh(),
        scratch_types=[
            pltpu.VMEM((CPW, CHUNK), jnp.int32),
            pltpu.VMEM((CHUNK, D), jnp.float32),
            pltpu.VMEM_SHARED((NPAD, D), jnp.float32),
        ],
    )(dst2d, ones128, z128)


GRP = 16
NGRP = CPW // GRP


def _edge_body(hs_hbm, src_hbm, dst_hbm, z128_hbm, out_hbm,
               sia_v, sib_v, dia_v, dib_v, ga, gb, sa, sb, sia, sib, acc):
    c = lax.axis_index("c")
    s = lax.axis_index("s")
    wid = c * 16 + s
    base = wid * CPW
    pltpu.sync_copy(z128_hbm, acc.at[pl.ds(s * ROWS_PT, ROWS_PT)])

    pltpu.async_copy(src_hbm.at[pl.ds(base, GRP)], sia_v, sia)
    pltpu.async_copy(dst_hbm.at[pl.ds(base, GRP)], dia_v, sia)
    plsc.subcore_barrier()

    for g in range(NGRP):
        p, q = g % 2, (g + 1) % 2
        sem = sia if p == 0 else sib
        srcv = sia_v if p == 0 else sib_v
        dstv = dia_v if p == 0 else dib_v
        gb_ = base + g * GRP
        pltpu.make_async_copy(src_hbm.at[pl.ds(gb_, GRP)], srcv, sem).wait()
        pltpu.make_async_copy(dst_hbm.at[pl.ds(gb_, GRP)], dstv, sem).wait()
        if g + 1 < NGRP:
            nsem = sia if q == 0 else sib
            nsrc = sia_v if q == 0 else sib_v
            ndst = dia_v if q == 0 else dib_v
            nb = base + (g + 1) * GRP
            pltpu.async_copy(src_hbm.at[pl.ds(nb, GRP)], nsrc, nsem)
            pltpu.async_copy(dst_hbm.at[pl.ds(nb, GRP)], ndst, nsem)

        pltpu.async_copy(hs_hbm.at[srcv.at[0]], ga, sa)

        def body(i, carry, srcv=srcv, dstv=dstv):
            j = 2 * i
            pltpu.make_async_copy(hs_hbm.at[srcv.at[j]], ga, sa).wait()
            pltpu.async_copy(hs_hbm.at[srcv.at[j + 1]], gb, sb)
            pltpu.sync_copy(ga, acc.at[dstv.at[j]], add=True)
            pltpu.make_async_copy(hs_hbm.at[srcv.at[j + 1]], gb, sb).wait()
            j2 = jnp.minimum(j + 2, GRP - 1)
            pltpu.async_copy(hs_hbm.at[srcv.at[j2]], ga, sa)
            pltpu.sync_copy(gb, acc.at[dstv.at[j + 1]], add=True)
            return carry

        lax.fori_loop(0, GRP // 2, body, 0)
        pltpu.make_async_copy(hs_hbm.at[srcv.at[0]], ga, sa).wait()

    plsc.subcore_barrier()
    pltpu.sync_copy(acc.at[pl.ds(s * ROWS_PT, ROWS_PT)],
                    out_hbm.at[c, pl.ds(s * ROWS_PT, ROWS_PT)])


def _edge_sc(hs, src2d, dst2d, z128):
    return pl.kernel(
        _edge_body,
        out_type=jax.ShapeDtypeStruct((2, NPAD, D), jnp.float32),
        mesh=_mesh(),
        scratch_types=[
            pltpu.VMEM((GRP, CHUNK), jnp.int32),
            pltpu.VMEM((GRP, CHUNK), jnp.int32),
            pltpu.VMEM((GRP, CHUNK), jnp.int32),
            pltpu.VMEM((GRP, CHUNK), jnp.int32),
            pltpu.VMEM((CHUNK, D), jnp.float32),
            pltpu.VMEM((CHUNK, D), jnp.float32),
            pltpu.SemaphoreType.DMA,
            pltpu.SemaphoreType.DMA,
            pltpu.SemaphoreType.DMA,
            pltpu.SemaphoreType.DMA,
            pltpu.VMEM_SHARED((NPAD, D), jnp.float32),
        ],
    )(hs, src2d, dst2d, z128)


def _prep_body(x_ref, w_ref, d0_ref, d1_ref, hs_ref, di_ref):
    deg = d0_ref[...] + d1_ref[...] + 1.0
    di = lax.rsqrt(deg)
    hs_ref[...] = jnp.dot(x_ref[...], w_ref[...],
                          preferred_element_type=jnp.float32) * di
    di_ref[...] = di


def _prep_tc(x, w1, d0, d1):
    return pl.pallas_call(
        _prep_body,
        grid=(GRID,),
        in_specs=[
            pl.BlockSpec((BLK, D), lambda i: (i, 0)),
            pl.BlockSpec((D, D), lambda i: (0, 0)),
            pl.BlockSpec((BLK, D), lambda i: (i, 0)),
            pl.BlockSpec((BLK, D), lambda i: (i, 0)),
        ],
        out_specs=[
            pl.BlockSpec((BLK, D), lambda i: (i, 0)),
            pl.BlockSpec((BLK, D), lambda i: (i, 0)),
        ],
        out_shape=[
            jax.ShapeDtypeStruct((NPAD, D), jnp.float32),
            jax.ShapeDtypeStruct((NPAD, D), jnp.float32),
        ],
    )(x, w1, d0, d1)


def _mid_body(a0_ref, a1_ref, hs_ref, di_ref, b_ref, w_ref, out_ref):
    di = di_ref[...]
    h = di * (a0_ref[...] + a1_ref[...] + hs_ref[...]) + b_ref[...]
    h = jnp.maximum(h, 0.0)
    out_ref[...] = jnp.dot(h, w_ref[...],
                           preferred_element_type=jnp.float32) * di


def _mid_tc(a0, a1, hs, di, b1, w2):
    return pl.pallas_call(
        _mid_body,
        grid=(GRID,),
        in_specs=[
            pl.BlockSpec((BLK, D), lambda i: (i, 0)),
            pl.BlockSpec((BLK, D), lambda i: (i, 0)),
            pl.BlockSpec((BLK, D), lambda i: (i, 0)),
            pl.BlockSpec((BLK, D), lambda i: (i, 0)),
            pl.BlockSpec((1, D), lambda i: (0, 0)),
            pl.BlockSpec((D, D), lambda i: (0, 0)),
        ],
        out_specs=pl.BlockSpec((BLK, D), lambda i: (i, 0)),
        out_shape=jax.ShapeDtypeStruct((NPAD, D), jnp.float32),
    )(a0, a1, hs, di, b1, w2)


def _final_body(a0_ref, a1_ref, hs_ref, di_ref, b_ref, h_ref, p_ref):
    i = pl.program_id(0)
    h = di_ref[...] * (a0_ref[...] + a1_ref[...] + hs_ref[...]) + b_ref[...]
    h = jnp.maximum(h, 0.0)
    h_ref[...] = h
    row = lax.broadcasted_iota(jnp.int32, (BLK, D), 0) + i * BLK
    hm = jnp.where(row < N, h, -jnp.inf)
    bm = jnp.max(hm, axis=0, keepdims=True)

    @pl.when(i == 0)
    def _():
        p_ref[...] = bm

    @pl.when(i > 0)
    def _():
        p_ref[...] = jnp.maximum(p_ref[...], bm)


def _final_tc(a0, a1, hs, di, b2):
    return pl.pallas_call(
        _final_body,
        grid=(GRID,),
        in_specs=[
            pl.BlockSpec((BLK, D), lambda i: (i, 0)),
            pl.BlockSpec((BLK, D), lambda i: (i, 0)),
            pl.BlockSpec((BLK, D), lambda i: (i, 0)),
            pl.BlockSpec((BLK, D), lambda i: (i, 0)),
            pl.BlockSpec((1, D), lambda i: (0, 0)),
        ],
        out_specs=[
            pl.BlockSpec((BLK, D), lambda i: (i, 0)),
            pl.BlockSpec((1, D), lambda i: (0, 0)),
        ],
        out_shape=[
            jax.ShapeDtypeStruct((NPAD, D), jnp.float32),
            jax.ShapeDtypeStruct((1, D), jnp.float32),
        ],
    )(a0, a1, hs, di, b2)


def kernel(x, edge_index, W1, b1, W2, b2):
    src = edge_index[0].astype(jnp.int32)
    dst = edge_index[1].astype(jnp.int32)
    pad = (N + (jnp.arange(EPAD - E, dtype=jnp.int32) % (NPAD - N)))
    src2d = jnp.concatenate([src, pad]).reshape(NW * CPW, CHUNK)
    dst2d = jnp.concatenate([dst, pad]).reshape(NW * CPW, CHUNK)
    xp = jnp.zeros((NPAD, D), jnp.float32).at[:N].set(x)

    ones128 = jnp.ones((CHUNK, D), jnp.float32)
    z128 = jnp.zeros((ROWS_PT, D), jnp.float32)
    b1r = b1.reshape(1, D)
    b2r = b2.reshape(1, D)

    degs = _deg_sc(dst2d, ones128, z128)
    hs1, dinv = _prep_tc(xp, W1, degs[0], degs[1])
    acc1 = _edge_sc(hs1, src2d, dst2d, z128)
    hs2 = _mid_tc(acc1[0], acc1[1], hs1, dinv, b1r, W2)
    acc2 = _edge_sc(hs2, src2d, dst2d, z128)
    h2, p = _final_tc(acc2[0], acc2[1], hs2, dinv, b2r)
    return (h2[:N], p)


# speedup vs baseline: 1.1734x; 1.0018x over previous
import functools

import jax
import jax.numpy as jnp
from jax import lax
from jax.experimental import pallas as pl
from jax.experimental.pallas import tpu as pltpu
from jax.experimental.pallas import tpu_sc as plsc

N = 10000
D = 128
NPAD = 10240
E = 320000
NW = 32
CHUNK = 128
CPW = 80
EPW = CPW * CHUNK
EPAD = NW * EPW
ROWS_PT = NPAD // 16
BLK = 512
GRID = NPAD // BLK

_mesh = functools.partial(
    plsc.VectorSubcoreMesh, core_axis_name="c", subcore_axis_name="s")


def _deg_body(dst_hbm, ones_hbm, z128_hbm, out_hbm, dstv, onev, sem, acc):
    c = lax.axis_index("c")
    s = lax.axis_index("s")
    wid = c * 16 + s
    pltpu.sync_copy(z128_hbm, acc.at[pl.ds(s * ROWS_PT, ROWS_PT)])
    pltpu.sync_copy(dst_hbm.at[pl.ds(wid * CPW, CPW)], dstv)
    pltpu.sync_copy(ones_hbm, onev)
    plsc.subcore_barrier()

    def body(b, carry):
        j = 8 * b
        for k in range(8):
            pltpu.async_copy(onev, acc.at[dstv.at[j + k]], sem, add=True)
        for k in range(8):
            pltpu.make_async_copy(onev, acc.at[dstv.at[j + k]], sem).wait()
        return carry

    lax.fori_loop(0, CPW // 8, body, 0)
    plsc.subcore_barrier()
    pltpu.sync_copy(acc.at[pl.ds(s * ROWS_PT, ROWS_PT)],
                    out_hbm.at[c, pl.ds(s * ROWS_PT, ROWS_PT)])


def _deg_sc(dst2d, ones128, z128):
    return pl.kernel(
        _deg_body,
        out_type=jax.ShapeDtypeStruct((2, NPAD, D), jnp.float32),
        mesh=_mesh(),
        scratch_types=[
            pltpu.VMEM((CPW, CHUNK), jnp.int32),
            pltpu.VMEM((CHUNK, D), jnp.float32),
            pltpu.SemaphoreType.DMA,
            pltpu.VMEM_SHARED((NPAD, D), jnp.float32),
        ],
    )(dst2d, ones128, z128)


GRP = 16
NGRP = CPW // GRP


def _edge_body(hs_hbm, src_hbm, dst_hbm, z128_hbm, out_hbm,
               sia_v, sib_v, dia_v, dib_v, ga, gb, sa, sb, ssa, ssb,
               sia, sib, acc):
    c = lax.axis_index("c")
    s = lax.axis_index("s")
    wid = c * 16 + s
    base = wid * CPW
    pltpu.sync_copy(z128_hbm, acc.at[pl.ds(s * ROWS_PT, ROWS_PT)])

    pltpu.async_copy(src_hbm.at[pl.ds(base, GRP)], sia_v, sia)
    pltpu.async_copy(dst_hbm.at[pl.ds(base, GRP)], dia_v, sia)
    plsc.subcore_barrier()

    for g in range(NGRP):
        p, q = g % 2, (g + 1) % 2
        sem = sia if p == 0 else sib
        srcv = sia_v if p == 0 else sib_v
        dstv = dia_v if p == 0 else dib_v
        gb_ = base + g * GRP
        pltpu.make_async_copy(src_hbm.at[pl.ds(gb_, GRP)], srcv, sem).wait()
        pltpu.make_async_copy(dst_hbm.at[pl.ds(gb_, GRP)], dstv, sem).wait()
        if g + 1 < NGRP:
            nsem = sia if q == 0 else sib
            nsrc = sia_v if q == 0 else sib_v
            ndst = dia_v if q == 0 else dib_v
            nb = base + (g + 1) * GRP
            pltpu.async_copy(src_hbm.at[pl.ds(nb, GRP)], nsrc, nsem)
            pltpu.async_copy(dst_hbm.at[pl.ds(nb, GRP)], ndst, nsem)

        def g_start(buf, sem_, j):
            pltpu.async_copy(hs_hbm.at[srcv.at[j]], buf, sem_)

        def g_wait(buf, sem_):
            pltpu.make_async_copy(hs_hbm.at[srcv.at[0]], buf, sem_).wait()

        def s_start(buf, sem_, j):
            pltpu.async_copy(buf, acc.at[dstv.at[j]], sem_, add=True)

        def s_wait(buf, sem_):
            pltpu.make_async_copy(buf, acc.at[dstv.at[0]], sem_).wait()

        g_start(ga, sa, 0)
        g_wait(ga, sa)
        s_start(ga, ssa, 0)
        g_start(gb, sb, 1)
        g_wait(gb, sb)
        s_start(gb, ssb, 1)
        s_wait(ga, ssa)
        g_start(ga, sa, 2)

        def body(i, carry, srcv=srcv, dstv=dstv):
            j = 2 * i
            g_wait(ga, sa)
            s_start(ga, ssa, j)
            s_wait(gb, ssb)
            g_start(gb, sb, j + 1)
            g_wait(gb, sb)
            s_start(gb, ssb, j + 1)
            s_wait(ga, ssa)
            g_start(ga, sa, jnp.minimum(j + 2, GRP - 1))
            return carry

        lax.fori_loop(1, GRP // 2, body, 0)
        s_wait(gb, ssb)
        g_wait(ga, sa)

    plsc.subcore_barrier()
    pltpu.sync_copy(acc.at[pl.ds(s * ROWS_PT, ROWS_PT)],
                    out_hbm.at[c, pl.ds(s * ROWS_PT, ROWS_PT)])


def _edge_sc(hs, src2d, dst2d, z128):
    return pl.kernel(
        _edge_body,
        out_type=jax.ShapeDtypeStruct((2, NPAD, D), jnp.float32),
        mesh=_mesh(),
        scratch_types=[
            pltpu.VMEM((GRP, CHUNK), jnp.int32),
            pltpu.VMEM((GRP, CHUNK), jnp.int32),
            pltpu.VMEM((GRP, CHUNK), jnp.int32),
            pltpu.VMEM((GRP, CHUNK), jnp.int32),
            pltpu.VMEM((CHUNK, D), jnp.float32),
            pltpu.VMEM((CHUNK, D), jnp.float32),
            pltpu.SemaphoreType.DMA,
            pltpu.SemaphoreType.DMA,
            pltpu.SemaphoreType.DMA,
            pltpu.SemaphoreType.DMA,
            pltpu.SemaphoreType.DMA,
            pltpu.SemaphoreType.DMA,
            pltpu.VMEM_SHARED((NPAD, D), jnp.float32),
        ],
    )(hs, src2d, dst2d, z128)


def _prep_body(x_ref, w_ref, d0_ref, d1_ref, hs_ref, di_ref):
    deg = d0_ref[...] + d1_ref[...] + 1.0
    di = lax.rsqrt(deg)
    hs_ref[...] = jnp.dot(x_ref[...], w_ref[...],
                          preferred_element_type=jnp.float32) * di
    di_ref[...] = di


def _prep_tc(x, w1, d0, d1):
    return pl.pallas_call(
        _prep_body,
        grid=(GRID,),
        in_specs=[
            pl.BlockSpec((BLK, D), lambda i: (i, 0)),
            pl.BlockSpec((D, D), lambda i: (0, 0)),
            pl.BlockSpec((BLK, D), lambda i: (i, 0)),
            pl.BlockSpec((BLK, D), lambda i: (i, 0)),
        ],
        out_specs=[
            pl.BlockSpec((BLK, D), lambda i: (i, 0)),
            pl.BlockSpec((BLK, D), lambda i: (i, 0)),
        ],
        out_shape=[
            jax.ShapeDtypeStruct((NPAD, D), jnp.float32),
            jax.ShapeDtypeStruct((NPAD, D), jnp.float32),
        ],
    )(x, w1, d0, d1)


def _mid_body(a0_ref, a1_ref, hs_ref, di_ref, b_ref, w_ref, out_ref):
    di = di_ref[...]
    h = di * (a0_ref[...] + a1_ref[...] + hs_ref[...]) + b_ref[...]
    h = jnp.maximum(h, 0.0)
    out_ref[...] = jnp.dot(h, w_ref[...],
                           preferred_element_type=jnp.float32) * di


def _mid_tc(a0, a1, hs, di, b1, w2):
    return pl.pallas_call(
        _mid_body,
        grid=(GRID,),
        in_specs=[
            pl.BlockSpec((BLK, D), lambda i: (i, 0)),
            pl.BlockSpec((BLK, D), lambda i: (i, 0)),
            pl.BlockSpec((BLK, D), lambda i: (i, 0)),
            pl.BlockSpec((BLK, D), lambda i: (i, 0)),
            pl.BlockSpec((1, D), lambda i: (0, 0)),
            pl.BlockSpec((D, D), lambda i: (0, 0)),
        ],
        out_specs=pl.BlockSpec((BLK, D), lambda i: (i, 0)),
        out_shape=jax.ShapeDtypeStruct((NPAD, D), jnp.float32),
    )(a0, a1, hs, di, b1, w2)


def _final_body(a0_ref, a1_ref, hs_ref, di_ref, b_ref, h_ref, p_ref):
    i = pl.program_id(0)
    h = di_ref[...] * (a0_ref[...] + a1_ref[...] + hs_ref[...]) + b_ref[...]
    h = jnp.maximum(h, 0.0)
    h_ref[...] = h
    row = lax.broadcasted_iota(jnp.int32, (BLK, D), 0) + i * BLK
    hm = jnp.where(row < N, h, -jnp.inf)
    bm = jnp.max(hm, axis=0, keepdims=True)

    @pl.when(i == 0)
    def _():
        p_ref[...] = bm

    @pl.when(i > 0)
    def _():
        p_ref[...] = jnp.maximum(p_ref[...], bm)


def _final_tc(a0, a1, hs, di, b2):
    return pl.pallas_call(
        _final_body,
        grid=(GRID,),
        in_specs=[
            pl.BlockSpec((BLK, D), lambda i: (i, 0)),
            pl.BlockSpec((BLK, D), lambda i: (i, 0)),
            pl.BlockSpec((BLK, D), lambda i: (i, 0)),
            pl.BlockSpec((BLK, D), lambda i: (i, 0)),
            pl.BlockSpec((1, D), lambda i: (0, 0)),
        ],
        out_specs=[
            pl.BlockSpec((BLK, D), lambda i: (i, 0)),
            pl.BlockSpec((1, D), lambda i: (0, 0)),
        ],
        out_shape=[
            jax.ShapeDtypeStruct((NPAD, D), jnp.float32),
            jax.ShapeDtypeStruct((1, D), jnp.float32),
        ],
    )(a0, a1, hs, di, b2)


def kernel(x, edge_index, W1, b1, W2, b2):
    src = edge_index[0].astype(jnp.int32)
    dst = edge_index[1].astype(jnp.int32)
    pad = (N + (jnp.arange(EPAD - E, dtype=jnp.int32) % (NPAD - N)))
    src2d = jnp.concatenate([src, pad]).reshape(NW * CPW, CHUNK)
    dst2d = jnp.concatenate([dst, pad]).reshape(NW * CPW, CHUNK)
    xp = jnp.zeros((NPAD, D), jnp.float32).at[:N].set(x)

    ones128 = jnp.ones((CHUNK, D), jnp.float32)
    z128 = jnp.zeros((ROWS_PT, D), jnp.float32)
    b1r = b1.reshape(1, D)
    b2r = b2.reshape(1, D)

    degs = _deg_sc(dst2d, ones128, z128)
    hs1, dinv = _prep_tc(xp, W1, degs[0], degs[1])
    acc1 = _edge_sc(hs1, src2d, dst2d, z128)
    hs2 = _mid_tc(acc1[0], acc1[1], hs1, dinv, b1r, W2)
    acc2 = _edge_sc(hs2, src2d, dst2d, z128)
    h2, p = _final_tc(acc2[0], acc2[1], hs2, dinv, b2r)
    return (h2[:N], p)


# speedup vs baseline: 1.3652x; 1.1634x over previous
import functools

import jax
import jax.numpy as jnp
from jax import lax
from jax.experimental import pallas as pl
from jax.experimental.pallas import tpu as pltpu
from jax.experimental.pallas import tpu_sc as plsc

N = 10000
D = 128
NPAD = 10240
E = 320000
NW = 32
CHUNK = 128
CPW = 80
EPW = CPW * CHUNK
EPAD = NW * EPW
ROWS_PT = NPAD // 16
BLK = 512
GRID = NPAD // BLK

_mesh = functools.partial(
    plsc.VectorSubcoreMesh, core_axis_name="c", subcore_axis_name="s")


def _deg_body(dst_hbm, out_hbm, dstv, degv, sumb, outv, stage):
    c = lax.axis_index("c")
    s = lax.axis_index("s")
    wid = c * 16 + s
    pltpu.sync_copy(dst_hbm.at[pl.ds(wid * CPW, CPW)], dstv)

    z16 = jnp.zeros((16,), jnp.float32)

    def zbody(t, carry):
        degv[pl.ds(16 * t, 16)] = z16
        return carry

    lax.fori_loop(0, NPAD // 16, zbody, 0)

    ones16 = jnp.full((16,), 1.0, jnp.float32)

    def body(j, carry):
        for k in range(CHUNK // 16):
            iv = dstv[j, pl.ds(16 * k, 16)]
            plsc.addupdate_scatter(degv, [iv], ones16)
        return carry

    lax.fori_loop(0, CPW, body, 0)

    pltpu.sync_copy(degv, stage.at[s])
    plsc.subcore_barrier()
    pltpu.sync_copy(stage.at[:, pl.ds(s * ROWS_PT, ROWS_PT)], sumb)

    def rbody(p, carry):
        a = sumb[0, pl.ds(16 * p, 16)]
        for r in range(1, 16):
            a = a + sumb[r, pl.ds(16 * p, 16)]
        outv[pl.ds(16 * p, 16)] = a
        return carry

    lax.fori_loop(0, ROWS_PT // 16, rbody, 0)
    pltpu.sync_copy(outv, out_hbm.at[c, pl.ds(s * ROWS_PT, ROWS_PT)])


def _deg_sc(dst2d):
    return pl.kernel(
        _deg_body,
        out_type=jax.ShapeDtypeStruct((2, NPAD), jnp.float32),
        mesh=_mesh(),
        compiler_params=pltpu.CompilerParams(needs_layout_passes=False),
        scratch_types=[
            pltpu.VMEM((CPW, CHUNK), jnp.int32),
            pltpu.VMEM((NPAD,), jnp.float32),
            pltpu.VMEM((16, ROWS_PT), jnp.float32),
            pltpu.VMEM((ROWS_PT,), jnp.float32),
            pltpu.VMEM_SHARED((16, NPAD), jnp.float32),
        ],
    )(dst2d)


GRP = 16
NGRP = CPW // GRP


def _edge_body(hs_hbm, src_hbm, dst_hbm, z128_hbm, out_hbm,
               sia_v, sib_v, dia_v, dib_v, ga, gb, sa, sb, ssa, ssb,
               sia, sib, acc):
    c = lax.axis_index("c")
    s = lax.axis_index("s")
    wid = c * 16 + s
    base = wid * CPW
    pltpu.sync_copy(z128_hbm, acc.at[pl.ds(s * ROWS_PT, ROWS_PT)])

    pltpu.async_copy(src_hbm.at[pl.ds(base, GRP)], sia_v, sia)
    pltpu.async_copy(dst_hbm.at[pl.ds(base, GRP)], dia_v, sia)
    plsc.subcore_barrier()

    for g in range(NGRP):
        p, q = g % 2, (g + 1) % 2
        sem = sia if p == 0 else sib
        srcv = sia_v if p == 0 else sib_v
        dstv = dia_v if p == 0 else dib_v
        gb_ = base + g * GRP
        pltpu.make_async_copy(src_hbm.at[pl.ds(gb_, GRP)], srcv, sem).wait()
        pltpu.make_async_copy(dst_hbm.at[pl.ds(gb_, GRP)], dstv, sem).wait()
        if g + 1 < NGRP:
            nsem = sia if q == 0 else sib
            nsrc = sia_v if q == 0 else sib_v
            ndst = dia_v if q == 0 else dib_v
            nb = base + (g + 1) * GRP
            pltpu.async_copy(src_hbm.at[pl.ds(nb, GRP)], nsrc, nsem)
            pltpu.async_copy(dst_hbm.at[pl.ds(nb, GRP)], ndst, nsem)

        def g_start(buf, sem_, j):
            pltpu.async_copy(hs_hbm.at[srcv.at[j]], buf, sem_)

        def g_wait(buf, sem_):
            pltpu.make_async_copy(hs_hbm.at[srcv.at[0]], buf, sem_).wait()

        def s_start(buf, sem_, j):
            pltpu.async_copy(buf, acc.at[dstv.at[j]], sem_, add=True)

        def s_wait(buf, sem_):
            pltpu.make_async_copy(buf, acc.at[dstv.at[0]], sem_).wait()

        g_start(ga, sa, 0)
        g_wait(ga, sa)
        s_start(ga, ssa, 0)
        g_start(gb, sb, 1)
        g_wait(gb, sb)
        s_start(gb, ssb, 1)
        s_wait(ga, ssa)
        g_start(ga, sa, 2)

        def body(i, carry, srcv=srcv, dstv=dstv):
            j = 2 * i
            g_wait(ga, sa)
            s_start(ga, ssa, j)
            s_wait(gb, ssb)
            g_start(gb, sb, j + 1)
            g_wait(gb, sb)
            s_start(gb, ssb, j + 1)
            s_wait(ga, ssa)
            g_start(ga, sa, jnp.minimum(j + 2, GRP - 1))
            return carry

        lax.fori_loop(1, GRP // 2, body, 0)
        s_wait(gb, ssb)
        g_wait(ga, sa)

    plsc.subcore_barrier()
    pltpu.sync_copy(acc.at[pl.ds(s * ROWS_PT, ROWS_PT)],
                    out_hbm.at[c, pl.ds(s * ROWS_PT, ROWS_PT)])


def _edge_sc(hs, src2d, dst2d, z128):
    return pl.kernel(
        _edge_body,
        out_type=jax.ShapeDtypeStruct((2, NPAD, D), jnp.float32),
        mesh=_mesh(),
        scratch_types=[
            pltpu.VMEM((GRP, CHUNK), jnp.int32),
            pltpu.VMEM((GRP, CHUNK), jnp.int32),
            pltpu.VMEM((GRP, CHUNK), jnp.int32),
            pltpu.VMEM((GRP, CHUNK), jnp.int32),
            pltpu.VMEM((CHUNK, D), jnp.float32),
            pltpu.VMEM((CHUNK, D), jnp.float32),
            pltpu.SemaphoreType.DMA,
            pltpu.SemaphoreType.DMA,
            pltpu.SemaphoreType.DMA,
            pltpu.SemaphoreType.DMA,
            pltpu.SemaphoreType.DMA,
            pltpu.SemaphoreType.DMA,
            pltpu.VMEM_SHARED((NPAD, D), jnp.float32),
        ],
    )(hs, src2d, dst2d, z128)


def _prep_body(x_ref, w_ref, d_ref, hs_ref, di_ref):
    di = lax.rsqrt(d_ref[...] + 1.0)
    hs_ref[...] = jnp.dot(x_ref[...], w_ref[...],
                          preferred_element_type=jnp.float32) * di
    di_ref[...] = di


def _prep_tc(x, w1, d):
    return pl.pallas_call(
        _prep_body,
        grid=(GRID,),
        in_specs=[
            pl.BlockSpec((BLK, D), lambda i: (i, 0)),
            pl.BlockSpec((D, D), lambda i: (0, 0)),
            pl.BlockSpec((BLK, D), lambda i: (i, 0)),
        ],
        out_specs=[
            pl.BlockSpec((BLK, D), lambda i: (i, 0)),
            pl.BlockSpec((BLK, D), lambda i: (i, 0)),
        ],
        out_shape=[
            jax.ShapeDtypeStruct((NPAD, D), jnp.float32),
            jax.ShapeDtypeStruct((NPAD, D), jnp.float32),
        ],
    )(x, w1, d)


def _mid_body(a0_ref, a1_ref, hs_ref, di_ref, b_ref, w_ref, out_ref):
    di = di_ref[...]
    h = di * (a0_ref[...] + a1_ref[...] + hs_ref[...]) + b_ref[...]
    h = jnp.maximum(h, 0.0)
    out_ref[...] = jnp.dot(h, w_ref[...],
                           preferred_element_type=jnp.float32) * di


def _mid_tc(a0, a1, hs, di, b1, w2):
    return pl.pallas_call(
        _mid_body,
        grid=(GRID,),
        in_specs=[
            pl.BlockSpec((BLK, D), lambda i: (i, 0)),
            pl.BlockSpec((BLK, D), lambda i: (i, 0)),
            pl.BlockSpec((BLK, D), lambda i: (i, 0)),
            pl.BlockSpec((BLK, D), lambda i: (i, 0)),
            pl.BlockSpec((1, D), lambda i: (0, 0)),
            pl.BlockSpec((D, D), lambda i: (0, 0)),
        ],
        out_specs=pl.BlockSpec((BLK, D), lambda i: (i, 0)),
        out_shape=jax.ShapeDtypeStruct((NPAD, D), jnp.float32),
    )(a0, a1, hs, di, b1, w2)


def _final_body(a0_ref, a1_ref, hs_ref, di_ref, b_ref, h_ref, p_ref):
    i = pl.program_id(0)
    h = di_ref[...] * (a0_ref[...] + a1_ref[...] + hs_ref[...]) + b_ref[...]
    h = jnp.maximum(h, 0.0)
    h_ref[...] = h
    row = lax.broadcasted_iota(jnp.int32, (BLK, D), 0) + i * BLK
    hm = jnp.where(row < N, h, -jnp.inf)
    bm = jnp.max(hm, axis=0, keepdims=True)

    @pl.when(i == 0)
    def _():
        p_ref[...] = bm

    @pl.when(i > 0)
    def _():
        p_ref[...] = jnp.maximum(p_ref[...], bm)


def _final_tc(a0, a1, hs, di, b2):
    return pl.pallas_call(
        _final_body,
        grid=(GRID,),
        in_specs=[
            pl.BlockSpec((BLK, D), lambda i: (i, 0)),
            pl.BlockSpec((BLK, D), lambda i: (i, 0)),
            pl.BlockSpec((BLK, D), lambda i: (i, 0)),
            pl.BlockSpec((BLK, D), lambda i: (i, 0)),
            pl.BlockSpec((1, D), lambda i: (0, 0)),
        ],
        out_specs=[
            pl.BlockSpec((BLK, D), lambda i: (i, 0)),
            pl.BlockSpec((1, D), lambda i: (0, 0)),
        ],
        out_shape=[
            jax.ShapeDtypeStruct((NPAD, D), jnp.float32),
            jax.ShapeDtypeStruct((1, D), jnp.float32),
        ],
    )(a0, a1, hs, di, b2)


def kernel(x, edge_index, W1, b1, W2, b2):
    src = edge_index[0].astype(jnp.int32)
    dst = edge_index[1].astype(jnp.int32)
    pad = (N + (jnp.arange(EPAD - E, dtype=jnp.int32) % (NPAD - N)))
    src2d = jnp.concatenate([src, pad]).reshape(NW * CPW, CHUNK)
    dst2d = jnp.concatenate([dst, pad]).reshape(NW * CPW, CHUNK)
    xp = jnp.zeros((NPAD, D), jnp.float32).at[:N].set(x)

    z128 = jnp.zeros((ROWS_PT, D), jnp.float32)
    b1r = b1.reshape(1, D)
    b2r = b2.reshape(1, D)

    degs = _deg_sc(dst2d)
    degb = jnp.broadcast_to((degs[0] + degs[1])[:, None], (NPAD, D))
    hs1, dinv = _prep_tc(xp, W1, degb)
    acc1 = _edge_sc(hs1, src2d, dst2d, z128)
    hs2 = _mid_tc(acc1[0], acc1[1], hs1, dinv, b1r, W2)
    acc2 = _edge_sc(hs2, src2d, dst2d, z128)
    h2, p = _final_tc(acc2[0], acc2[1], hs2, dinv, b2r)
    return (h2[:N], p)


# speedup vs baseline: 1.3687x; 1.0026x over previous
import functools

import jax
import jax.numpy as jnp
from jax import lax
from jax.experimental import pallas as pl
from jax.experimental.pallas import tpu as pltpu
from jax.experimental.pallas import tpu_sc as plsc

N = 10000
D = 128
NPAD = 10240
E = 320000
NW = 32
CHUNK = 128
CPW = 80
EPW = CPW * CHUNK
EPAD = NW * EPW
ROWS_PT = NPAD // 16
BLK = 512
GRID = NPAD // BLK

_mesh = functools.partial(
    plsc.VectorSubcoreMesh, core_axis_name="c", subcore_axis_name="s")


def _deg_body(dst_hbm, out_hbm, dstv, degv, sumb, outv, stage):
    c = lax.axis_index("c")
    s = lax.axis_index("s")
    wid = c * 16 + s
    pltpu.sync_copy(dst_hbm.at[pl.ds(wid * CPW, CPW)], dstv)

    z16 = jnp.zeros((16,), jnp.float32)

    def zbody(t, carry):
        degv[pl.ds(16 * t, 16)] = z16
        return carry

    lax.fori_loop(0, NPAD // 16, zbody, 0)

    ones16 = jnp.full((16,), 1.0, jnp.float32)

    def body(j, carry):
        for k in range(CHUNK // 16):
            iv = dstv[j, pl.ds(16 * k, 16)]
            plsc.addupdate_scatter(degv, [iv], ones16)
        return carry

    lax.fori_loop(0, CPW, body, 0)

    pltpu.sync_copy(degv, stage.at[s])
    plsc.subcore_barrier()
    pltpu.sync_copy(stage.at[:, pl.ds(s * ROWS_PT, ROWS_PT)], sumb)

    def rbody(p, carry):
        a = sumb[0, pl.ds(16 * p, 16)]
        for r in range(1, 16):
            a = a + sumb[r, pl.ds(16 * p, 16)]
        outv[pl.ds(16 * p, 16)] = a
        return carry

    lax.fori_loop(0, ROWS_PT // 16, rbody, 0)
    pltpu.sync_copy(outv, out_hbm.at[c, pl.ds(s * ROWS_PT, ROWS_PT)])


def _deg_sc(dst2d):
    return pl.kernel(
        _deg_body,
        out_type=jax.ShapeDtypeStruct((2, NPAD), jnp.float32),
        mesh=_mesh(),
        compiler_params=pltpu.CompilerParams(needs_layout_passes=False,
                                             skip_device_barrier=True),
        scratch_types=[
            pltpu.VMEM((CPW, CHUNK), jnp.int32),
            pltpu.VMEM((NPAD,), jnp.float32),
            pltpu.VMEM((16, ROWS_PT), jnp.float32),
            pltpu.VMEM((ROWS_PT,), jnp.float32),
            pltpu.VMEM_SHARED((16, NPAD), jnp.float32),
        ],
    )(dst2d)


GRP = 16
NGRP = CPW // GRP


def _edge_body(hs_hbm, src_hbm, dst_hbm, z128_hbm, out_hbm,
               sia_v, sib_v, dia_v, dib_v, ga, gb, sa, sb, ssa, ssb,
               sia, sib, acc):
    c = lax.axis_index("c")
    s = lax.axis_index("s")
    wid = c * 16 + s
    base = wid * CPW
    pltpu.sync_copy(z128_hbm, acc.at[pl.ds(s * ROWS_PT, ROWS_PT)])

    pltpu.async_copy(src_hbm.at[pl.ds(base, GRP)], sia_v, sia)
    pltpu.async_copy(dst_hbm.at[pl.ds(base, GRP)], dia_v, sia)
    plsc.subcore_barrier()

    for g in range(NGRP):
        p, q = g % 2, (g + 1) % 2
        sem = sia if p == 0 else sib
        srcv = sia_v if p == 0 else sib_v
        dstv = dia_v if p == 0 else dib_v
        gb_ = base + g * GRP
        pltpu.make_async_copy(src_hbm.at[pl.ds(gb_, GRP)], srcv, sem).wait()
        pltpu.make_async_copy(dst_hbm.at[pl.ds(gb_, GRP)], dstv, sem).wait()
        if g + 1 < NGRP:
            nsem = sia if q == 0 else sib
            nsrc = sia_v if q == 0 else sib_v
            ndst = dia_v if q == 0 else dib_v
            nb = base + (g + 1) * GRP
            pltpu.async_copy(src_hbm.at[pl.ds(nb, GRP)], nsrc, nsem)
            pltpu.async_copy(dst_hbm.at[pl.ds(nb, GRP)], ndst, nsem)

        def g_start(buf, sem_, j):
            pltpu.async_copy(hs_hbm.at[srcv.at[j]], buf, sem_)

        def g_wait(buf, sem_):
            pltpu.make_async_copy(hs_hbm.at[srcv.at[0]], buf, sem_).wait()

        def s_start(buf, sem_, j):
            pltpu.async_copy(buf, acc.at[dstv.at[j]], sem_, add=True)

        def s_wait(buf, sem_):
            pltpu.make_async_copy(buf, acc.at[dstv.at[0]], sem_).wait()

        g_start(ga, sa, 0)
        g_wait(ga, sa)
        s_start(ga, ssa, 0)
        g_start(gb, sb, 1)
        g_wait(gb, sb)
        s_start(gb, ssb, 1)
        s_wait(ga, ssa)
        g_start(ga, sa, 2)

        def body(i, carry, srcv=srcv, dstv=dstv):
            j = 2 * i
            g_wait(ga, sa)
            s_start(ga, ssa, j)
            s_wait(gb, ssb)
            g_start(gb, sb, j + 1)
            g_wait(gb, sb)
            s_start(gb, ssb, j + 1)
            s_wait(ga, ssa)
            g_start(ga, sa, jnp.minimum(j + 2, GRP - 1))
            return carry

        lax.fori_loop(1, GRP // 2, body, 0)
        s_wait(gb, ssb)
        g_wait(ga, sa)

    plsc.subcore_barrier()
    pltpu.sync_copy(acc.at[pl.ds(s * ROWS_PT, ROWS_PT)],
                    out_hbm.at[c, pl.ds(s * ROWS_PT, ROWS_PT)])


def _edge_sc(hs, src2d, dst2d, z128):
    return pl.kernel(
        _edge_body,
        out_type=jax.ShapeDtypeStruct((2, NPAD, D), jnp.float32),
        mesh=_mesh(),
        compiler_params=pltpu.CompilerParams(skip_device_barrier=True),
        scratch_types=[
            pltpu.VMEM((GRP, CHUNK), jnp.int32),
            pltpu.VMEM((GRP, CHUNK), jnp.int32),
            pltpu.VMEM((GRP, CHUNK), jnp.int32),
            pltpu.VMEM((GRP, CHUNK), jnp.int32),
            pltpu.VMEM((CHUNK, D), jnp.float32),
            pltpu.VMEM((CHUNK, D), jnp.float32),
            pltpu.SemaphoreType.DMA,
            pltpu.SemaphoreType.DMA,
            pltpu.SemaphoreType.DMA,
            pltpu.SemaphoreType.DMA,
            pltpu.SemaphoreType.DMA,
            pltpu.SemaphoreType.DMA,
            pltpu.VMEM_SHARED((NPAD, D), jnp.float32),
        ],
    )(hs, src2d, dst2d, z128)


def _prep_body(x_ref, w_ref, d_ref, hs_ref, di_ref):
    di = lax.rsqrt(d_ref[...] + 1.0)
    hs_ref[...] = jnp.dot(x_ref[...], w_ref[...],
                          preferred_element_type=jnp.float32) * di
    di_ref[...] = di


def _prep_tc(x, w1, d):
    return pl.pallas_call(
        _prep_body,
        grid=(GRID,),
        in_specs=[
            pl.BlockSpec((BLK, D), lambda i: (i, 0)),
            pl.BlockSpec((D, D), lambda i: (0, 0)),
            pl.BlockSpec((BLK, D), lambda i: (i, 0)),
        ],
        out_specs=[
            pl.BlockSpec((BLK, D), lambda i: (i, 0)),
            pl.BlockSpec((BLK, D), lambda i: (i, 0)),
        ],
        out_shape=[
            jax.ShapeDtypeStruct((NPAD, D), jnp.float32),
            jax.ShapeDtypeStruct((NPAD, D), jnp.float32),
        ],
    )(x, w1, d)


def _mid_body(a0_ref, a1_ref, hs_ref, di_ref, b_ref, w_ref, out_ref):
    di = di_ref[...]
    h = di * (a0_ref[...] + a1_ref[...] + hs_ref[...]) + b_ref[...]
    h = jnp.maximum(h, 0.0)
    out_ref[...] = jnp.dot(h, w_ref[...],
                           preferred_element_type=jnp.float32) * di


def _mid_tc(a0, a1, hs, di, b1, w2):
    return pl.pallas_call(
        _mid_body,
        grid=(GRID,),
        in_specs=[
            pl.BlockSpec((BLK, D), lambda i: (i, 0)),
            pl.BlockSpec((BLK, D), lambda i: (i, 0)),
            pl.BlockSpec((BLK, D), lambda i: (i, 0)),
            pl.BlockSpec((BLK, D), lambda i: (i, 0)),
            pl.BlockSpec((1, D), lambda i: (0, 0)),
            pl.BlockSpec((D, D), lambda i: (0, 0)),
        ],
        out_specs=pl.BlockSpec((BLK, D), lambda i: (i, 0)),
        out_shape=jax.ShapeDtypeStruct((NPAD, D), jnp.float32),
    )(a0, a1, hs, di, b1, w2)


def _final_body(a0_ref, a1_ref, hs_ref, di_ref, b_ref, h_ref, p_ref):
    i = pl.program_id(0)
    h = di_ref[...] * (a0_ref[...] + a1_ref[...] + hs_ref[...]) + b_ref[...]
    h = jnp.maximum(h, 0.0)
    h_ref[...] = h
    row = lax.broadcasted_iota(jnp.int32, (BLK, D), 0) + i * BLK
    hm = jnp.where(row < N, h, -jnp.inf)
    bm = jnp.max(hm, axis=0, keepdims=True)

    @pl.when(i == 0)
    def _():
        p_ref[...] = bm

    @pl.when(i > 0)
    def _():
        p_ref[...] = jnp.maximum(p_ref[...], bm)


def _final_tc(a0, a1, hs, di, b2):
    return pl.pallas_call(
        _final_body,
        grid=(GRID,),
        in_specs=[
            pl.BlockSpec((BLK, D), lambda i: (i, 0)),
            pl.BlockSpec((BLK, D), lambda i: (i, 0)),
            pl.BlockSpec((BLK, D), lambda i: (i, 0)),
            pl.BlockSpec((BLK, D), lambda i: (i, 0)),
            pl.BlockSpec((1, D), lambda i: (0, 0)),
        ],
        out_specs=[
            pl.BlockSpec((BLK, D), lambda i: (i, 0)),
            pl.BlockSpec((1, D), lambda i: (0, 0)),
        ],
        out_shape=[
            jax.ShapeDtypeStruct((NPAD, D), jnp.float32),
            jax.ShapeDtypeStruct((1, D), jnp.float32),
        ],
    )(a0, a1, hs, di, b2)


def kernel(x, edge_index, W1, b1, W2, b2):
    src = edge_index[0].astype(jnp.int32)
    dst = edge_index[1].astype(jnp.int32)
    pad = (N + (jnp.arange(EPAD - E, dtype=jnp.int32) % (NPAD - N)))
    src2d = jnp.concatenate([src, pad]).reshape(NW * CPW, CHUNK)
    dst2d = jnp.concatenate([dst, pad]).reshape(NW * CPW, CHUNK)
    xp = jnp.zeros((NPAD, D), jnp.float32).at[:N].set(x)

    z128 = jnp.zeros((ROWS_PT, D), jnp.float32)
    b1r = b1.reshape(1, D)
    b2r = b2.reshape(1, D)

    degs = _deg_sc(dst2d)
    degb = jnp.broadcast_to((degs[0] + degs[1])[:, None], (NPAD, D))
    hs1, dinv = _prep_tc(xp, W1, degb)
    acc1 = _edge_sc(hs1, src2d, dst2d, z128)
    hs2 = _mid_tc(acc1[0], acc1[1], hs1, dinv, b1r, W2)
    acc2 = _edge_sc(hs2, src2d, dst2d, z128)
    h2, p = _final_tc(acc2[0], acc2[1], hs2, dinv, b2r)
    return (h2[:N], p)
